# baseline (speedup 1.0000x reference)
import jax
import jax.numpy as jnp
from jax.experimental import pallas as pl
from jax.experimental.pallas import tpu as pltpu

_BLOCK = 4096


def _gate_kernel(x_ref, w_ref, b_ref, logits_ref, weights_ref):
    b_col = b_ref[...].reshape(w_ref.shape[0], 1)
    logits = jax.lax.dot_general(
        w_ref[...], x_ref[...], (((1,), (1,)), ((), ())),
        preferred_element_type=jnp.float32) + b_col
    sl = pl.ds((pl.program_id(0) % 2) * _BLOCK, _BLOCK)
    logits_ref[:, sl] = logits
    m = jnp.max(logits, axis=0, keepdims=True)
    e = jnp.exp(logits - m)
    weights_ref[:, sl] = e / jnp.sum(e, axis=0, keepdims=True)


def kernel(x, W, b):
    n_tokens, dim = x.shape
    n_experts = W.shape[0]
    b2 = b.reshape(1, n_experts)
    grid = (n_tokens // _BLOCK,)
    logits_t, weights_t = pl.pallas_call(
        _gate_kernel,
        grid=grid,
        in_specs=[
            pl.BlockSpec((_BLOCK, dim), lambda i: (i, 0)),
            pl.BlockSpec((n_experts, dim), lambda i: (0, 0)),
            pl.BlockSpec((1, n_experts), lambda i: (0, 0)),
        ],
        out_specs=[
            pl.BlockSpec((n_experts, 2 * _BLOCK), lambda i: (0, i // 2)),
            pl.BlockSpec((n_experts, 2 * _BLOCK), lambda i: (0, i // 2)),
        ],
        out_shape=[
            jax.ShapeDtypeStruct((n_experts, n_tokens), jnp.float32),
            jax.ShapeDtypeStruct((n_experts, n_tokens), jnp.float32),
        ],
        compiler_params=pltpu.CompilerParams(
            dimension_semantics=("parallel",)),
    )(x, W, b2)
    tau = max(1.0, 1e-06)
    return (weights_t.T, logits_t.T, tau)


# speedup vs baseline: 1.0104x; 1.0104x over previous
import jax
import jax.numpy as jnp
from jax.experimental import pallas as pl
from jax.experimental.pallas import tpu as pltpu

_BLOCK = 4096


def _gate_kernel(x_ref, w_ref, b_ref, logits_ref, weights_ref):
    b_col = b_ref[...].reshape(w_ref.shape[0], 1)
    logits = jax.lax.dot_general(
        w_ref[...], x_ref[...], (((1,), (1,)), ((), ())),
        preferred_element_type=jnp.float32) + b_col
    logits_ref[...] = logits
    m = jnp.max(logits, axis=0, keepdims=True)
    e = jnp.exp(logits - m)
    weights_ref[...] = e / jnp.sum(e, axis=0, keepdims=True)


def kernel(x, W, b):
    n_tokens, dim = x.shape
    n_experts = W.shape[0]
    b2 = b.reshape(1, n_experts)
    grid = (n_tokens // _BLOCK,)
    logits_t, weights_t = pl.pallas_call(
        _gate_kernel,
        grid=grid,
        in_specs=[
            pl.BlockSpec((_BLOCK, dim), lambda i: (i, 0)),
            pl.BlockSpec((n_experts, dim), lambda i: (0, 0)),
            pl.BlockSpec((1, n_experts), lambda i: (0, 0)),
        ],
        out_specs=[
            pl.BlockSpec((n_experts, _BLOCK), lambda i: (0, i)),
            pl.BlockSpec((n_experts, _BLOCK), lambda i: (0, i)),
        ],
        out_shape=[
            jax.ShapeDtypeStruct((n_experts, n_tokens), jnp.float32),
            jax.ShapeDtypeStruct((n_experts, n_tokens), jnp.float32),
        ],
        compiler_params=pltpu.CompilerParams(
            dimension_semantics=("parallel",)),
    )(x, W, b2)
    tau = max(1.0, 1e-06)
    return (weights_t.T, logits_t.T, tau)
